# baseline (speedup 1.0000x reference)
import functools

import jax
import jax.numpy as jnp
from jax import lax
from jax.experimental import pallas as pl
from jax.experimental.pallas import tpu as pltpu
from jax.experimental.pallas import tpu_sc as plsc

_S, _K, _D = 8, 512, 8
_NB = 2048
_CH = 128


def _argmin_body(z_ref, w_ref, idx_ref):
    z = z_ref[...]
    w = w_ref[...]
    scores = []
    for s in range(_S):
        zs = z[:, s * _D:(s + 1) * _D]
        ws = w[s]
        cross2 = lax.dot_general(
            zs, 2.0 * ws, (((1,), (1,)), ((), ())),
            preferred_element_type=jnp.float32)
        ww = jnp.sum(ws * ws, axis=1)[None, :]
        scores.append(cross2 - ww)
    cols = [jnp.argmax(sc, axis=1).astype(jnp.int32)[:, None] + s * _K
            for s, sc in enumerate(scores)]
    idx_ref[...] = jnp.concatenate(cols, axis=1)


def _tc_indices(z2, weight, half, nh):
    off = half * (nh // _NB)
    return pl.pallas_call(
        _argmin_body,
        grid=(nh // _NB,),
        in_specs=[
            pl.BlockSpec((_NB, _S * _D), lambda i, off=off: (i + off, 0)),
            pl.BlockSpec((_S, _K, _D), lambda i: (0, 0, 0)),
        ],
        out_specs=pl.BlockSpec((_NB, _S), lambda i: (i, 0)),
        out_shape=jax.ShapeDtypeStruct((nh, _S), jnp.int32),
    )(z2, weight)


def _gather_body(table_hbm, idx_hbm, out_hbm, idx_v, rows_v, sem):
    nch = idx_v.shape[0]
    bpw = nch * _CH
    wid = lax.axis_index("s") * 2 + lax.axis_index("c")
    pltpu.sync_copy(idx_hbm.at[pl.ds(wid * nch, nch)], idx_v)

    def grp(g, carry):
        h = [pltpu.async_copy(table_hbm.at[idx_v.at[g * 8 + j]],
                              rows_v.at[pl.ds((g * 8 + j) * _CH, _CH)], sem)
             for j in range(8)]
        for c in h:
            c.wait()
        return carry
    lax.fori_loop(0, nch // 8, grp, 0)
    pltpu.sync_copy(rows_v, out_hbm.at[pl.ds(wid * bpw, bpw)])


def _sc_gather(table, idx2):
    nw = 32
    b = idx2.shape[0] * idx2.shape[1]
    nch = b // (nw * _CH)
    fn = functools.partial(
        pl.kernel,
        mesh=plsc.VectorSubcoreMesh(core_axis_name="c", subcore_axis_name="s"),
        out_type=jax.ShapeDtypeStruct((b, _D), jnp.float32),
        scratch_types=[
            pltpu.VMEM((nch, _CH), jnp.int32),
            pltpu.VMEM((nch * _CH, _D), jnp.float32),
            pltpu.SemaphoreType.DMA,
        ],
        compiler_params=pltpu.CompilerParams(use_tc_tiling_on_sc=False),
    )(_gather_body)
    return fn(table, idx2)


def kernel(z, weight):
    zshape = z.shape
    z2 = z.reshape(-1, _S * _D)
    n = z2.shape[0]
    table = weight.reshape(_S * _K, _D)
    outs = []
    for h in range(2):
        gidx = _tc_indices(z2, weight, h, n // 2)
        outs.append(_sc_gather(table, gidx.reshape(-1, _CH)))
    return jnp.concatenate(outs, axis=0).reshape(zshape)


# speedup vs baseline: 1.6688x; 1.6688x over previous
import functools

import jax
import jax.numpy as jnp
from jax import lax
from jax.experimental import pallas as pl
from jax.experimental.pallas import tpu as pltpu
from jax.experimental.pallas import tpu_sc as plsc

_S, _K, _D = 8, 512, 8
_NB = 2048
_CH = 128


def _argmin_body(z_ref, w_ref, idx_ref):
    z = z_ref[...]
    w = w_ref[...]
    scores = []
    for s in range(_S):
        zs = z[:, s * _D:(s + 1) * _D]
        ws = w[s]
        cross2 = lax.dot_general(
            zs, 2.0 * ws, (((1,), (1,)), ((), ())),
            preferred_element_type=jnp.float32)
        ww = jnp.sum(ws * ws, axis=1)[None, :]
        scores.append(cross2 - ww)
    cols = [jnp.argmax(sc, axis=1).astype(jnp.int32)[:, None] + s * _K
            for s, sc in enumerate(scores)]
    idx_ref[...] = jnp.concatenate(cols, axis=1)


def _tc_indices(z2, weight):
    n = z2.shape[0]
    return pl.pallas_call(
        _argmin_body,
        grid=(n // _NB,),
        in_specs=[
            pl.BlockSpec((_NB, _S * _D), lambda i: (i, 0)),
            pl.BlockSpec((_S, _K, _D), lambda i: (0, 0, 0)),
        ],
        out_specs=pl.BlockSpec((_NB, _S), lambda i: (i, 0)),
        out_shape=jax.ShapeDtypeStruct((n, _S), jnp.int32),
    )(z2, weight)


def _gather_body(table_hbm, idx_hbm, out_hbm, idx_v, rows_v, sem):
    nch = idx_v.shape[0]
    bpw = nch * _CH
    wid = lax.axis_index("s") * 2 + lax.axis_index("c")
    pltpu.sync_copy(idx_hbm.at[pl.ds(wid * nch, nch)], idx_v)

    def grp(g, carry):
        h = [pltpu.async_copy(table_hbm.at[idx_v.at[g * 8 + j]],
                              rows_v.at[pl.ds((g * 8 + j) * _CH, _CH)], sem)
             for j in range(8)]
        for c in h:
            c.wait()
        return carry
    lax.fori_loop(0, nch // 8, grp, 0)
    pltpu.sync_copy(rows_v, out_hbm.at[pl.ds(wid * bpw, bpw)])


def _sc_gather(table, idx2):
    nw = 32
    b = idx2.shape[0] * idx2.shape[1]
    nch = b // (nw * _CH)
    fn = functools.partial(
        pl.kernel,
        mesh=plsc.VectorSubcoreMesh(core_axis_name="c", subcore_axis_name="s"),
        out_type=jax.ShapeDtypeStruct((b, _D), jnp.float32),
        scratch_types=[
            pltpu.VMEM((nch, _CH), jnp.int32),
            pltpu.VMEM((nch * _CH, _D), jnp.float32),
            pltpu.SemaphoreType.DMA,
        ],
        compiler_params=pltpu.CompilerParams(use_tc_tiling_on_sc=False),
    )(_gather_body)
    return fn(table, idx2)


def kernel(z, weight):
    zshape = z.shape
    z2 = z.reshape(-1, _S * _D)
    gidx = _tc_indices(z2, weight)
    table = weight.reshape(_S * _K, _D)
    idx2 = gidx.reshape(-1, _CH)
    out = _sc_gather(table, idx2)
    return out.reshape(zshape)
